# baseline (speedup 1.0000x reference)
import functools

import jax
import jax.numpy as jnp
from jax import lax
from jax.experimental import pallas as pl
from jax.experimental.pallas import tpu as pltpu
from jax.experimental.pallas import tpu_sc as plsc

_THRESH = 0.35667494393873245
_C0 = -0.8472978603872036
_LN2 = 0.6931471805599453

_N = 16 * 512 * 512
_NMIN = int(_N * 0.15)
_NW = 32
_PER_W = _N // _NW
_CHUNK = 8192
_NCHUNK = _PER_W // _CHUNK
_L = 16
_FLUSH = 32
_NBLK = _CHUNK // (_L * _FLUSH)


def _vlog(p):
    bits = lax.bitcast_convert_type(p, jnp.int32)
    ex = ((bits >> 23) & 0xFF) - 127
    m = lax.bitcast_convert_type((bits & 0x7FFFFF) | 0x3F800000, jnp.float32)
    r = (m - 1.0) / (m + 1.0)
    r2 = r * r
    poly = 1.0 + r2 * (1.0 / 3.0 + r2 * (1.0 / 5.0 + r2 * (
        1.0 / 7.0 + r2 * (1.0 / 9.0 + r2 * (1.0 / 11.0 + r2 * (1.0 / 13.0))))))
    return ex.astype(jnp.float32) * _LN2 + (2.0 * r) * poly


def _accum_chunk(xbuf, ybuf, off0, vsum, vcnt, mask_fn):
    def blk_body(blk, carry):
        vsum, vcnt = carry
        base = off0 + blk * (_L * _FLUSH)
        prod = jnp.full((_L,), 1.0, jnp.float32)
        for j in range(_FLUSH):
            o = base + j * _L
            x = xbuf[pl.ds(o, _L)]
            y = ybuf[pl.ds(o, _L)]
            t = (1.0 - 2.0 * y) * x
            r = jnp.maximum(t, 0.0)
            e = jnp.exp(jnp.minimum(t, -t))
            msk = mask_fn(t, r, e)
            prod = prod * jnp.where(msk, 1.0 + e, 1.0)
            vsum = vsum + jnp.where(msk, r, 0.0)
            vcnt = vcnt + jnp.where(msk, 1.0, 0.0)
        vsum = vsum + _vlog(prod)
        return (vsum, vcnt)

    return lax.fori_loop(0, _NBLK, blk_body, (vsum, vcnt))


def _stream_body(x_hbm, y_hbm, out_hbm, xbuf, ybuf, ostage, sem0, sem1,
                 mask_fn):
    wid = lax.axis_index("s") * 2 + lax.axis_index("c")
    base = wid * _PER_W

    def start(g, slot, sem):
        pltpu.async_copy(x_hbm.at[pl.ds(base + g * _CHUNK, _CHUNK)],
                         xbuf.at[pl.ds(slot * _CHUNK, _CHUNK)], sem)
        pltpu.async_copy(y_hbm.at[pl.ds(base + g * _CHUNK, _CHUNK)],
                         ybuf.at[pl.ds(slot * _CHUNK, _CHUNK)], sem)

    def wait(slot, sem):
        pltpu.make_async_copy(x_hbm.at[pl.ds(base, _CHUNK)],
                              xbuf.at[pl.ds(slot * _CHUNK, _CHUNK)],
                              sem).wait()
        pltpu.make_async_copy(y_hbm.at[pl.ds(base, _CHUNK)],
                              ybuf.at[pl.ds(slot * _CHUNK, _CHUNK)],
                              sem).wait()

    start(0, 0, sem0)
    start(1, 1, sem1)

    def g_body(gp, carry):
        vsum, vcnt = carry
        for b, sem in ((0, sem0), (1, sem1)):
            g = gp * 2 + b
            wait(b, sem)
            vsum, vcnt = _accum_chunk(xbuf, ybuf, b * _CHUNK, vsum, vcnt,
                                      mask_fn)

            @pl.when(g + 2 < _NCHUNK)
            def _():
                start(g + 2, b, sem)
        return (vsum, vcnt)

    vsum, vcnt = lax.fori_loop(
        0, _NCHUNK // 2, g_body,
        (jnp.zeros((_L,), jnp.float32), jnp.zeros((_L,), jnp.float32)))
    ostage[pl.ds(0, _L)] = vsum
    ostage[pl.ds(_L, _L)] = vcnt
    pltpu.sync_copy(ostage, out_hbm.at[wid])


_OUT = jax.ShapeDtypeStruct((_NW, 2 * _L), jnp.float32)


@functools.cache
def _fast_kernel():
    mesh = plsc.VectorSubcoreMesh(core_axis_name="c", subcore_axis_name="s")

    @functools.partial(
        pl.kernel, mesh=mesh, out_type=_OUT,
        scratch_types=[
            pltpu.VMEM((2 * _CHUNK,), jnp.float32),
            pltpu.VMEM((2 * _CHUNK,), jnp.float32),
            pltpu.VMEM((2 * _L,), jnp.float32),
            pltpu.SemaphoreType.DMA,
            pltpu.SemaphoreType.DMA,
        ])
    def k(x_hbm, y_hbm, out_hbm, xbuf, ybuf, ostage, sem0, sem1):
        _stream_body(x_hbm, y_hbm, out_hbm, xbuf, ybuf, ostage, sem0, sem1,
                     lambda t, r, e: t > _C0)

    return k


@functools.cache
def _thr_kernel():
    mesh = plsc.VectorSubcoreMesh(core_axis_name="c", subcore_axis_name="s")

    @functools.partial(
        pl.kernel, mesh=mesh, out_type=_OUT,
        scratch_types=[
            pltpu.VMEM((2 * _CHUNK,), jnp.float32),
            pltpu.VMEM((2 * _CHUNK,), jnp.float32),
            pltpu.VMEM((_L,), jnp.float32),
            pltpu.VMEM((2 * _L,), jnp.float32),
            pltpu.SemaphoreType.DMA,
            pltpu.SemaphoreType.DMA,
        ])
    def k(x_hbm, y_hbm, t_hbm, out_hbm, xbuf, ybuf, tbuf, ostage,
          sem0, sem1):
        pltpu.sync_copy(t_hbm, tbuf)
        thv = tbuf[...]
        _stream_body(x_hbm, y_hbm, out_hbm, xbuf, ybuf, ostage, sem0, sem1,
                     lambda t, r, e: e > (jnp.exp(thv - r) - 1.0))

    return k


def kernel(logits, labels):
    x = logits.reshape(-1)
    y = labels.reshape(-1)
    parts = _fast_kernel()(x, y)
    vsum = jnp.sum(parts[:, :_L])
    count = jnp.sum(parts[:, _L:])
    nminf = jnp.float32(_NMIN)

    def _masked(_):
        return vsum / count

    def _hard(_):
        def body(c):
            lo, hi = c
            mid = lo + (hi - lo) // 2
            tv = lax.bitcast_convert_type(mid, jnp.float32)
            p = _thr_kernel()(x, y, jnp.full((_L,), tv, jnp.float32))
            below = jnp.sum(p[:, _L:]) < nminf
            return (jnp.where(below, lo, mid + 1), jnp.where(below, mid, hi))

        lo, _ = lax.while_loop(lambda c: c[0] < c[1], body,
                               (jnp.int32(0), jnp.int32(0x7F7FFFFF)))
        v = lax.bitcast_convert_type(lo, jnp.float32)
        p = _thr_kernel()(x, y, jnp.full((_L,), v, jnp.float32))
        sum_gt = jnp.sum(p[:, :_L])
        cnt_gt = jnp.sum(p[:, _L:])
        return (sum_gt + (nminf - cnt_gt) * v) / nminf

    return lax.cond(count < nminf, _hard, _masked, 0)
